# baseline (speedup 1.0000x reference)
import jax
import jax.numpy as jnp
from jax.experimental import pallas as pl
from jax.experimental.pallas import tpu as pltpu
from jax.experimental.pallas import tpu_sc as plsc

_N = 16384
_D = 256
_K = 8192
_BN = 256
_NH = _N // 2
_NTH = _NH // _BN
_NT = _N // _BN
_GW = 128
_COMMIT = 0.25


def _dist_argmin_body(x_ref, x2_ref, e_ref, e2_ref, idx_ref, pcnt_ref):
    x = x_ref[...]
    e = e_ref[...]
    mm = jax.lax.dot_general(
        x, e, (((1,), (1,)), ((), ())),
        preferred_element_type=jnp.float32)
    dist = (x2_ref[...] + e2_ref[...]) - 2.0 * mm
    minval = jnp.min(dist, axis=1, keepdims=True)
    col = jax.lax.broadcasted_iota(jnp.int32, dist.shape, 1)
    idx = jnp.min(jnp.where(dist == minval, col, _K), axis=1)
    idx_ref[0, 0, :] = idx
    onehot = (idx[:, None] == col).astype(jnp.bfloat16)
    ones = jnp.ones((1, _BN), jnp.bfloat16)
    pcnt = jax.lax.dot_general(
        ones, onehot, (((1,), (0,)), ((), ())),
        preferred_element_type=jnp.float32)
    pcnt_ref[0, ...] = pcnt


def _tc_stage(x, x2, e, e2, interpret=False):
    return pl.pallas_call(
        _dist_argmin_body,
        grid=(_NTH,),
        in_specs=[
            pl.BlockSpec((_BN, _D), lambda i: (i, 0)),
            pl.BlockSpec((_BN, 1), lambda i: (i, 0)),
            pl.BlockSpec((_K, _D), lambda i: (0, 0)),
            pl.BlockSpec((1, _K), lambda i: (0, 0)),
        ],
        out_specs=[
            pl.BlockSpec((1, 1, _BN), lambda i: (i, 0, 0)),
            pl.BlockSpec((1, 1, _K), lambda i: (i, 0, 0)),
        ],
        out_shape=[
            jax.ShapeDtypeStruct((_NTH, 1, _BN), jnp.int32),
            jax.ShapeDtypeStruct((_NTH, 1, _K), jnp.float32),
        ],
        compiler_params=pltpu.CompilerParams(
            dimension_semantics=("parallel",)),
        interpret=interpret,
    )(x, x2, e, e2)


def _qst_loss_body(x_ref, q_ref, pcnt_ref, qst_ref, loss_ref, pplx_ref):
    i = pl.program_id(0)
    x = x_ref[...]
    q = q_ref[...]
    d = q - x
    qst_ref[...] = x + d
    m = jnp.sum(d * d, axis=1) * (1.0 / _D)
    loss_ref[0, 0, :] = (1.0 + _COMMIT) * m

    @pl.when(i == 0)
    def _pplx():
        cnt = jnp.sum(pcnt_ref[...], axis=0, keepdims=True)
        p = cnt * (1.0 / _N)
        ent = jnp.sum(p * jnp.log(p + 1e-10), axis=1, keepdims=True)
        pplx_ref[...] = jnp.exp(-ent)


def _qst_loss_stage(x, q, pcnt, interpret=False):
    return pl.pallas_call(
        _qst_loss_body,
        grid=(_NT,),
        in_specs=[
            pl.BlockSpec((_BN, _D), lambda i: (i, 0)),
            pl.BlockSpec((_BN, _D), lambda i: (i, 0)),
            pl.BlockSpec((_NT, _K), lambda i: (0, 0)),
        ],
        out_specs=[
            pl.BlockSpec((_BN, _D), lambda i: (i, 0)),
            pl.BlockSpec((1, 1, _BN), lambda i: (i, 0, 0)),
            pl.BlockSpec((1, 1), lambda i: (0, 0)),
        ],
        out_shape=[
            jax.ShapeDtypeStruct((_N, _D), jnp.float32),
            jax.ShapeDtypeStruct((_NT, 1, _BN), jnp.float32),
            jax.ShapeDtypeStruct((1, 1), jnp.float32),
        ],
        compiler_params=pltpu.CompilerParams(
            dimension_semantics=("arbitrary",)),
        interpret=interpret,
    )(x, q, pcnt)


def _sc_gather(e, idx, n_rows):
    idx2 = idx.reshape(1, n_rows)

    @pl.kernel(out_type=jax.ShapeDtypeStruct((n_rows, _D), jnp.float32),
               mesh=plsc.VectorSubcoreMesh(core_axis_name="core",
                                           subcore_axis_name="subcore"))
    def _gather_kernel(e_hbm, i_hbm, o_hbm):
        def body(i_vmem, o_vmem):
            pltpu.sync_copy(e_hbm.at[i_vmem.at[0]], o_vmem)

        pltpu.emit_pipeline(
            body,
            grid=(n_rows // _GW,),
            in_specs=[pl.BlockSpec((1, _GW), index_map=lambda i: (0, i))],
            out_specs=[pl.BlockSpec((_GW, _D), index_map=lambda i: (i, 0))],
            core_axis_name=("core", "subcore"),
            dimension_semantics=(pltpu.PARALLEL,),
        )(i_hbm, o_hbm)

    return _gather_kernel(e, idx2)


def kernel(flat_input, embedding_weight):
    x2 = jnp.sum(flat_input ** 2, axis=1, keepdims=True)
    e2 = jnp.sum(embedding_weight ** 2, axis=1).reshape(1, _K)
    x_lo, x_hi = flat_input[:_NH], flat_input[_NH:]
    x2_lo, x2_hi = x2[:_NH], x2[_NH:]
    idx_a, pcnt_a = _tc_stage(x_lo, x2_lo, embedding_weight, e2)
    q_a = _sc_gather(embedding_weight, idx_a.reshape(_NH), _NH)
    idx_b, pcnt_b = _tc_stage(x_hi, x2_hi, embedding_weight, e2)
    q_b = _sc_gather(embedding_weight, idx_b.reshape(_NH), _NH)
    indices = jnp.concatenate([idx_a.reshape(_NH), idx_b.reshape(_NH)])
    quantized = jnp.concatenate([q_a, q_b], axis=0)
    pcnt = jnp.concatenate(
        [pcnt_a.reshape(_NTH, _K), pcnt_b.reshape(_NTH, _K)], axis=0)
    quantized_st, loss3, pplx = _qst_loss_stage(flat_input, quantized, pcnt)
    loss = loss3.reshape(_N)
    perplexity = pplx[0, 0]
    return (quantized_st, loss, perplexity, indices)


# speedup vs baseline: 1.0782x; 1.0782x over previous
import jax
import jax.numpy as jnp
from jax.experimental import pallas as pl
from jax.experimental.pallas import tpu as pltpu
from jax.experimental.pallas import tpu_sc as plsc

_N = 16384
_D = 256
_K = 8192
_BN = 256
_NH = _N // 2
_NTH = _NH // _BN
_NT = _N // _BN
_GW = 128
_COMMIT = 0.25


def _dist_argmin_body(x_ref, x2_ref, e_ref, e2_ref, idx_ref, pcnt_ref):
    x = x_ref[...]
    e = e_ref[...]
    mm = jax.lax.dot_general(
        x, e, (((1,), (1,)), ((), ())),
        preferred_element_type=jnp.float32)
    dist = (x2_ref[...] + e2_ref[...]) - 2.0 * mm
    minval = jnp.min(dist, axis=1, keepdims=True)
    col = jax.lax.broadcasted_iota(jnp.int32, dist.shape, 1)
    idx = jnp.min(jnp.where(dist == minval, col, _K), axis=1)
    idx_ref[0, 0, :] = idx
    onehot = (idx[:, None] == col).astype(jnp.bfloat16)
    ones = jnp.ones((1, _BN), jnp.bfloat16)
    pcnt = jax.lax.dot_general(
        ones, onehot, (((1,), (0,)), ((), ())),
        preferred_element_type=jnp.float32)
    pcnt_ref[0, ...] = pcnt


def _tc_stage(x, x2, e, e2, interpret=False):
    return pl.pallas_call(
        _dist_argmin_body,
        grid=(_NT,),
        in_specs=[
            pl.BlockSpec((_BN, _D), lambda i: (i, 0)),
            pl.BlockSpec((_BN, 1), lambda i: (i, 0)),
            pl.BlockSpec((_K, _D), lambda i: (0, 0)),
            pl.BlockSpec((1, _K), lambda i: (0, 0)),
        ],
        out_specs=[
            pl.BlockSpec((1, 1, _BN), lambda i: (i, 0, 0)),
            pl.BlockSpec((1, 1, _K), lambda i: (i, 0, 0)),
        ],
        out_shape=[
            jax.ShapeDtypeStruct((_NT, 1, _BN), jnp.int32),
            jax.ShapeDtypeStruct((_NT, 1, _K), jnp.float32),
        ],
        compiler_params=pltpu.CompilerParams(
            dimension_semantics=("parallel",)),
        interpret=interpret,
    )(x, x2, e, e2)


def _qst_loss_body(x_ref, q_ref, pcnt_ref, qst_ref, loss_ref, pplx_ref):
    i = pl.program_id(0)
    x = x_ref[...]
    q = q_ref[...]
    d = q - x
    qst_ref[...] = x + d
    m = jnp.sum(d * d, axis=1) * (1.0 / _D)
    loss_ref[0, 0, :] = (1.0 + _COMMIT) * m

    @pl.when(i == 0)
    def _pplx():
        cnt = jnp.sum(pcnt_ref[...], axis=0, keepdims=True)
        p = cnt * (1.0 / _N)
        ent = jnp.sum(p * jnp.log(p + 1e-10), axis=1, keepdims=True)
        pplx_ref[...] = jnp.exp(-ent)


def _qst_loss_stage(x, q, pcnt, interpret=False):
    return pl.pallas_call(
        _qst_loss_body,
        grid=(_NT,),
        in_specs=[
            pl.BlockSpec((_BN, _D), lambda i: (i, 0)),
            pl.BlockSpec((_BN, _D), lambda i: (i, 0)),
            pl.BlockSpec((_NT, _K), lambda i: (0, 0)),
        ],
        out_specs=[
            pl.BlockSpec((_BN, _D), lambda i: (i, 0)),
            pl.BlockSpec((1, 1, _BN), lambda i: (i, 0, 0)),
            pl.BlockSpec((1, 1), lambda i: (0, 0)),
        ],
        out_shape=[
            jax.ShapeDtypeStruct((_N, _D), jnp.float32),
            jax.ShapeDtypeStruct((_NT, 1, _BN), jnp.float32),
            jax.ShapeDtypeStruct((1, 1), jnp.float32),
        ],
        compiler_params=pltpu.CompilerParams(
            dimension_semantics=("arbitrary",)),
        interpret=interpret,
    )(x, q, pcnt)


def _sc_gather(e, idx, n_rows):
    idx2 = idx.reshape(1, n_rows)

    @pl.kernel(out_type=jax.ShapeDtypeStruct((n_rows, _D), jnp.float32),
               mesh=plsc.VectorSubcoreMesh(core_axis_name="core",
                                           subcore_axis_name="subcore"))
    def _gather_kernel(e_hbm, i_hbm, o_hbm):
        def body(i_vmem, o_vmem):
            pltpu.sync_copy(e_hbm.at[i_vmem.at[0]], o_vmem)

        pltpu.emit_pipeline(
            body,
            grid=(n_rows // _GW,),
            in_specs=[pl.BlockSpec((1, _GW), index_map=lambda i: (0, i))],
            out_specs=[pl.BlockSpec((_GW, _D), index_map=lambda i: (i, 0))],
            core_axis_name=("core", "subcore"),
            dimension_semantics=(pltpu.PARALLEL,),
        )(i_hbm, o_hbm)

    return _gather_kernel(e, idx2)


def kernel(flat_input, embedding_weight):
    x2 = jnp.sum(flat_input ** 2, axis=1, keepdims=True)
    e2 = jnp.sum(embedding_weight ** 2, axis=1).reshape(1, _K)
    idx3, pcnt3 = _tc_stage(flat_input, x2, embedding_weight, e2)
    indices = idx3.reshape(_N)
    quantized = _sc_gather(embedding_weight, indices, _N)
    quantized_st, loss3, pplx = _qst_loss_stage(
        flat_input, quantized, pcnt3.reshape(_NT, _K))
    loss = loss3.reshape(_N)
    perplexity = pplx[0, 0]
    return (quantized_st, loss, perplexity, indices)


# speedup vs baseline: 1.2333x; 1.1438x over previous
import jax
import jax.numpy as jnp
from jax.experimental import pallas as pl
from jax.experimental.pallas import tpu as pltpu
from jax.experimental.pallas import tpu_sc as plsc

_N = 16384
_D = 256
_K = 8192
_BN = 512
_NH = _N // 2
_NTH = _NH // _BN
_NT = _N // _BN
_GW = 128
_COMMIT = 0.25


def _dist_argmin_body(x_ref, x2_ref, e_ref, e2_ref, idx_ref, pcnt_ref):
    x = x_ref[...]
    e = e_ref[...]
    mm = jax.lax.dot_general(
        x, e, (((1,), (1,)), ((), ())),
        preferred_element_type=jnp.float32)
    dist = (x2_ref[...] + e2_ref[...]) - 2.0 * mm
    minval = jnp.min(dist, axis=1, keepdims=True)
    col = jax.lax.broadcasted_iota(jnp.int32, dist.shape, 1)
    idx = jnp.min(jnp.where(dist == minval, col, _K), axis=1)
    idx_ref[0, 0, :] = idx
    onehot = (idx[:, None] == col).astype(jnp.bfloat16)
    ones = jnp.ones((1, _BN), jnp.bfloat16)
    pcnt = jax.lax.dot_general(
        ones, onehot, (((1,), (0,)), ((), ())),
        preferred_element_type=jnp.float32)
    pcnt_ref[0, ...] = pcnt


def _tc_stage(x, x2, e, e2, interpret=False):
    return pl.pallas_call(
        _dist_argmin_body,
        grid=(_NT,),
        in_specs=[
            pl.BlockSpec((_BN, _D), lambda i: (i, 0)),
            pl.BlockSpec((_BN, 1), lambda i: (i, 0)),
            pl.BlockSpec((_K, _D), lambda i: (0, 0)),
            pl.BlockSpec((1, _K), lambda i: (0, 0)),
        ],
        out_specs=[
            pl.BlockSpec((1, 1, _BN), lambda i: (i, 0, 0)),
            pl.BlockSpec((1, 1, _K), lambda i: (i, 0, 0)),
        ],
        out_shape=[
            jax.ShapeDtypeStruct((_NT, 1, _BN), jnp.int32),
            jax.ShapeDtypeStruct((_NT, 1, _K), jnp.float32),
        ],
        compiler_params=pltpu.CompilerParams(
            dimension_semantics=("parallel",)),
        interpret=interpret,
    )(x, x2, e, e2)


def _qst_loss_body(x_ref, q_ref, pcnt_ref, qst_ref, loss_ref, pplx_ref):
    i = pl.program_id(0)
    x = x_ref[...]
    q = q_ref[...]
    d = q - x
    qst_ref[...] = x + d
    m = jnp.sum(d * d, axis=1) * (1.0 / _D)
    loss_ref[0, 0, :] = (1.0 + _COMMIT) * m

    @pl.when(i == 0)
    def _pplx():
        cnt = jnp.sum(pcnt_ref[...], axis=0, keepdims=True)
        p = cnt * (1.0 / _N)
        ent = jnp.sum(p * jnp.log(p + 1e-10), axis=1, keepdims=True)
        pplx_ref[...] = jnp.exp(-ent)


def _qst_loss_stage(x, q, pcnt, interpret=False):
    return pl.pallas_call(
        _qst_loss_body,
        grid=(_NT,),
        in_specs=[
            pl.BlockSpec((_BN, _D), lambda i: (i, 0)),
            pl.BlockSpec((_BN, _D), lambda i: (i, 0)),
            pl.BlockSpec((_NT, _K), lambda i: (0, 0)),
        ],
        out_specs=[
            pl.BlockSpec((_BN, _D), lambda i: (i, 0)),
            pl.BlockSpec((1, 1, _BN), lambda i: (i, 0, 0)),
            pl.BlockSpec((1, 1), lambda i: (0, 0)),
        ],
        out_shape=[
            jax.ShapeDtypeStruct((_N, _D), jnp.float32),
            jax.ShapeDtypeStruct((_NT, 1, _BN), jnp.float32),
            jax.ShapeDtypeStruct((1, 1), jnp.float32),
        ],
        compiler_params=pltpu.CompilerParams(
            dimension_semantics=("arbitrary",)),
        interpret=interpret,
    )(x, q, pcnt)


def _sc_gather(e, idx, n_rows):
    idx2 = idx.reshape(1, n_rows)

    @pl.kernel(out_type=jax.ShapeDtypeStruct((n_rows, _D), jnp.float32),
               mesh=plsc.VectorSubcoreMesh(core_axis_name="core",
                                           subcore_axis_name="subcore"))
    def _gather_kernel(e_hbm, i_hbm, o_hbm):
        def body(i_vmem, o_vmem):
            pltpu.sync_copy(e_hbm.at[i_vmem.at[0]], o_vmem)

        pltpu.emit_pipeline(
            body,
            grid=(n_rows // _GW,),
            in_specs=[pl.BlockSpec((1, _GW), index_map=lambda i: (0, i))],
            out_specs=[pl.BlockSpec((_GW, _D), index_map=lambda i: (i, 0))],
            core_axis_name=("core", "subcore"),
            dimension_semantics=(pltpu.PARALLEL,),
        )(i_hbm, o_hbm)

    return _gather_kernel(e, idx2)


def kernel(flat_input, embedding_weight):
    x2 = jnp.sum(flat_input ** 2, axis=1, keepdims=True)
    e2 = jnp.sum(embedding_weight ** 2, axis=1).reshape(1, _K)
    idx3, pcnt3 = _tc_stage(flat_input, x2, embedding_weight, e2)
    indices = idx3.reshape(_N)
    quantized = _sc_gather(embedding_weight, indices, _N)
    quantized_st, loss3, pplx = _qst_loss_stage(
        flat_input, quantized, pcnt3.reshape(_NT, _K))
    loss = loss3.reshape(_N)
    perplexity = pplx[0, 0]
    return (quantized_st, loss, perplexity, indices)


# speedup vs baseline: 1.3160x; 1.0671x over previous
import jax
import jax.numpy as jnp
from jax.experimental import pallas as pl
from jax.experimental.pallas import tpu as pltpu
from jax.experimental.pallas import tpu_sc as plsc

_N = 16384
_D = 256
_K = 8192
_BN = 1024
_NH = _N // 2
_NTH = _NH // _BN
_NT = _N // _BN
_GW = 128
_COMMIT = 0.25


def _dist_argmin_body(x_ref, x2_ref, e_ref, e2_ref, idx_ref, pcnt_ref):
    x = x_ref[...]
    e = e_ref[...]
    mm = jax.lax.dot_general(
        x, e, (((1,), (1,)), ((), ())),
        preferred_element_type=jnp.float32)
    dist = (x2_ref[...] + e2_ref[...]) - 2.0 * mm
    minval = jnp.min(dist, axis=1, keepdims=True)
    col = jax.lax.broadcasted_iota(jnp.int32, dist.shape, 1)
    idx = jnp.min(jnp.where(dist == minval, col, _K), axis=1)
    idx_ref[0, 0, :] = idx
    onehot = (idx[:, None] == col).astype(jnp.bfloat16)
    ones = jnp.ones((1, _BN), jnp.bfloat16)
    pcnt = jax.lax.dot_general(
        ones, onehot, (((1,), (0,)), ((), ())),
        preferred_element_type=jnp.float32)
    pcnt_ref[0, ...] = pcnt


def _tc_stage(x, x2, e, e2, interpret=False):
    return pl.pallas_call(
        _dist_argmin_body,
        grid=(_NT,),
        in_specs=[
            pl.BlockSpec((_BN, _D), lambda i: (i, 0)),
            pl.BlockSpec((_BN, 1), lambda i: (i, 0)),
            pl.BlockSpec((_K, _D), lambda i: (0, 0)),
            pl.BlockSpec((1, _K), lambda i: (0, 0)),
        ],
        out_specs=[
            pl.BlockSpec((1, 1, _BN), lambda i: (i, 0, 0)),
            pl.BlockSpec((1, 1, _K), lambda i: (i, 0, 0)),
        ],
        out_shape=[
            jax.ShapeDtypeStruct((_NT, 1, _BN), jnp.int32),
            jax.ShapeDtypeStruct((_NT, 1, _K), jnp.float32),
        ],
        compiler_params=pltpu.CompilerParams(
            dimension_semantics=("parallel",)),
        interpret=interpret,
    )(x, x2, e, e2)


def _qst_loss_body(x_ref, q_ref, pcnt_ref, qst_ref, loss_ref, pplx_ref):
    i = pl.program_id(0)
    x = x_ref[...]
    q = q_ref[...]
    d = q - x
    qst_ref[...] = x + d
    m = jnp.sum(d * d, axis=1) * (1.0 / _D)
    loss_ref[0, 0, :] = (1.0 + _COMMIT) * m

    @pl.when(i == 0)
    def _pplx():
        cnt = jnp.sum(pcnt_ref[...], axis=0, keepdims=True)
        p = cnt * (1.0 / _N)
        ent = jnp.sum(p * jnp.log(p + 1e-10), axis=1, keepdims=True)
        pplx_ref[...] = jnp.exp(-ent)


def _qst_loss_stage(x, q, pcnt, interpret=False):
    return pl.pallas_call(
        _qst_loss_body,
        grid=(_NT,),
        in_specs=[
            pl.BlockSpec((_BN, _D), lambda i: (i, 0)),
            pl.BlockSpec((_BN, _D), lambda i: (i, 0)),
            pl.BlockSpec((_NT, _K), lambda i: (0, 0)),
        ],
        out_specs=[
            pl.BlockSpec((_BN, _D), lambda i: (i, 0)),
            pl.BlockSpec((1, 1, _BN), lambda i: (i, 0, 0)),
            pl.BlockSpec((1, 1), lambda i: (0, 0)),
        ],
        out_shape=[
            jax.ShapeDtypeStruct((_N, _D), jnp.float32),
            jax.ShapeDtypeStruct((_NT, 1, _BN), jnp.float32),
            jax.ShapeDtypeStruct((1, 1), jnp.float32),
        ],
        compiler_params=pltpu.CompilerParams(
            dimension_semantics=("arbitrary",)),
        interpret=interpret,
    )(x, q, pcnt)


def _sc_gather(e, idx, n_rows):
    idx2 = idx.reshape(1, n_rows)

    @pl.kernel(out_type=jax.ShapeDtypeStruct((n_rows, _D), jnp.float32),
               mesh=plsc.VectorSubcoreMesh(core_axis_name="core",
                                           subcore_axis_name="subcore"))
    def _gather_kernel(e_hbm, i_hbm, o_hbm):
        def body(i_vmem, o_vmem):
            pltpu.sync_copy(e_hbm.at[i_vmem.at[0]], o_vmem)

        pltpu.emit_pipeline(
            body,
            grid=(n_rows // _GW,),
            in_specs=[pl.BlockSpec((1, _GW), index_map=lambda i: (0, i))],
            out_specs=[pl.BlockSpec((_GW, _D), index_map=lambda i: (i, 0))],
            core_axis_name=("core", "subcore"),
            dimension_semantics=(pltpu.PARALLEL,),
        )(i_hbm, o_hbm)

    return _gather_kernel(e, idx2)


def kernel(flat_input, embedding_weight):
    x2 = jnp.sum(flat_input ** 2, axis=1, keepdims=True)
    e2 = jnp.sum(embedding_weight ** 2, axis=1).reshape(1, _K)
    idx3, pcnt3 = _tc_stage(flat_input, x2, embedding_weight, e2)
    indices = idx3.reshape(_N)
    quantized = _sc_gather(embedding_weight, indices, _N)
    quantized_st, loss3, pplx = _qst_loss_stage(
        flat_input, quantized, pcnt3.reshape(_NT, _K))
    loss = loss3.reshape(_N)
    perplexity = pplx[0, 0]
    return (quantized_st, loss, perplexity, indices)
